# baseline (speedup 1.0000x reference)
import functools

import jax
import jax.numpy as jnp
from jax import lax
from jax.experimental import pallas as pl
from jax.experimental.pallas import tpu as pltpu
from jax.experimental.pallas import tpu_sc as plsc

N = 10000
E = 320000
D_IN = 128
D = 16
D2 = 8
D_OUT = 7

NTILE = 16
NSC = 2
NW = NTILE * NSC

NPAD = 10240
RPT = NPAD // NTILE
EPAD = 327680
BATCH = 128
CROWS = 2048
NCH0 = 6
NCH1 = 4
ET0 = NCH0 * CROWS
ET1 = NCH1 * CROWS
BASE1 = NTILE * ET0
NB0 = ET0 // BATCH
NB1 = ET1 // BATCH

_MESH = plsc.VectorSubcoreMesh(core_axis_name="c", subcore_axis_name="s")


NB0D = 110
NB1D = 50
BASE1D = NTILE * NB0D * BATCH


@functools.partial(
    pl.kernel,
    out_type=jax.ShapeDtypeStruct((NSC, NPAD // 8, 128), jnp.float32),
    mesh=_MESH,
    scratch_types=[
        pltpu.VMEM((NB0D, BATCH), jnp.int32),
        pltpu.VMEM((BATCH,), jnp.float32),
        pltpu.VMEM((RPT,), jnp.float32),
        pltpu.VMEM((RPT // 8, 128), jnp.float32),
        pltpu.VMEM_SHARED((NPAD,), jnp.float32),
        pltpu.SemaphoreType.DMA,
        pltpu.SemaphoreType.DMA,
    ],
    compiler_params=pltpu.CompilerParams(needs_layout_passes=False),
)
def _sc_degree(dst0_hbm, dst1_hbm, ones_hbm, z1_hbm, deg_hbm, dst_v, ones_v,
               dv_v, dw_v, deg_s, sem, ssem):
    c = lax.axis_index("c")
    s = lax.axis_index("s")

    pltpu.sync_copy(ones_hbm, ones_v)
    pltpu.sync_copy(z1_hbm.at[pl.ds(s * RPT, RPT)],
                    deg_s.at[pl.ds(s * RPT, RPT)])

    def scatter_deg(nb):
        def issue(g, carry):
            pltpu.async_copy(ones_v, deg_s.at[dst_v.at[g]], ssem, add=True)
            return carry

        lax.fori_loop(0, nb, issue, 0)

        def drain(g, carry):
            pltpu.make_async_copy(ones_v, deg_s.at[dst_v.at[0]], ssem).wait()
            return carry

        lax.fori_loop(0, nb, drain, 0)

    @pl.when(c == 0)
    def _():
        pltpu.async_copy(dst0_hbm.at[s], dst_v, sem).wait()
        plsc.subcore_barrier()
        scatter_deg(NB0D)

    @pl.when(c == 1)
    def _():
        pltpu.async_copy(dst1_hbm.at[s], dst_v.at[pl.ds(0, NB1D)], sem).wait()
        plsc.subcore_barrier()
        scatter_deg(NB1D)

    plsc.subcore_barrier()

    pltpu.sync_copy(deg_s.at[pl.ds(s * RPT, RPT)], dv_v)

    def widen(j, carry):
        idx = jnp.broadcast_to(j, (16,)).astype(jnp.int32)
        dw_v[j // 8, pl.ds((j % 8) * 16, 16)] = plsc.load_gather(dv_v, [idx])
        return carry

    lax.fori_loop(0, RPT, widen, 0, unroll=4)
    pltpu.sync_copy(dw_v, deg_hbm.at[c, pl.ds(s * (RPT // 8), RPT // 8)])


def _make_prop(d):
    @functools.partial(
        pl.kernel,
        out_type=jax.ShapeDtypeStruct((NSC, NPAD, d), jnp.float32),
        mesh=_MESH,
        scratch_types=[
            pltpu.VMEM((NCH0, CROWS), jnp.int32),
            pltpu.VMEM((NCH0, CROWS), jnp.int32),
            pltpu.VMEM((CROWS, d), jnp.float32),
            pltpu.VMEM((CROWS, d), jnp.float32),
            pltpu.VMEM_SHARED((NPAD, d), jnp.float32),
            pltpu.VMEM_SHARED((NPAD, d), jnp.float32),
            pltpu.SemaphoreType.DMA,
            pltpu.SemaphoreType.DMA,
            pltpu.SemaphoreType.DMA,
        ],
        compiler_params=pltpu.CompilerParams(use_tc_tiling_on_sc=False),
    )
    def _sc_prop(g_hbm, src_hbm, dst_hbm, z2_hbm, p_hbm, src_v, dst_v,
                 rows0_v, rows1_v, acc_s, g_s, sem0, sem1, semg):
        c = lax.axis_index("c")
        s = lax.axis_index("s")

        cpg = pltpu.async_copy(g_hbm.at[pl.ds(s * RPT, RPT)],
                               g_s.at[pl.ds(s * RPT, RPT)], semg)
        pltpu.sync_copy(z2_hbm.at[pl.ds(s * RPT, RPT)],
                        acc_s.at[pl.ds(s * RPT, RPT)])

        rows = (rows0_v, rows1_v)
        sems = (sem0, sem1)

        def run(nch, base):
            for k in range(nch):
                pltpu.async_copy(
                    src_hbm.at[pl.ds(base + k * CROWS, CROWS)],
                    src_v.at[k], sem0)
                pltpu.async_copy(
                    dst_hbm.at[pl.ds(base + k * CROWS, CROWS)],
                    dst_v.at[k], sem1)
            for k in range(nch):
                pltpu.make_async_copy(
                    src_hbm.at[pl.ds(base, CROWS)], src_v.at[0], sem0).wait()
                pltpu.make_async_copy(
                    dst_hbm.at[pl.ds(base, CROWS)], dst_v.at[0], sem1).wait()
            cpg.wait()
            plsc.subcore_barrier()

            pltpu.async_copy(g_s.at[src_v.at[0]], rows0_v, sem0)
            for ci in range(nch):
                if ci + 1 < nch:
                    pltpu.async_copy(g_s.at[src_v.at[ci + 1]],
                                     rows[(ci + 1) % 2], sems[(ci + 1) % 2])
                pltpu.make_async_copy(g_s.at[src_v.at[ci]],
                                      rows[ci % 2], sems[ci % 2]).wait()
                pltpu.sync_copy(rows[ci % 2], acc_s.at[dst_v.at[ci]], add=True)

        @pl.when(c == 0)
        def _():
            run(NCH0, s * ET0)

        @pl.when(c == 1)
        def _():
            run(NCH1, BASE1 + s * ET1)

        plsc.subcore_barrier()

        pltpu.sync_copy(acc_s.at[pl.ds(s * RPT, RPT)],
                        p_hbm.at[c, pl.ds(s * RPT, RPT)])

    return _sc_prop


_sc_prop1 = _make_prop(D)


RBLK = 1024


PB = 128


def _tc1_body(x_ref, w_ref, deg_ref, g_ref, dis_ref):
    d = deg_ref[0] + deg_ref[1] + 1.0
    dis = lax.rsqrt(d)
    h = jnp.dot(x_ref[...], w_ref[...], preferred_element_type=jnp.float32,
                precision=lax.Precision.HIGHEST)
    g_ref[...] = dis * h
    dis_ref[...] = dis


_tc1 = pl.pallas_call(
    _tc1_body,
    grid=(NPAD // RBLK,),
    in_specs=[
        pl.BlockSpec((PB, 8 * D_IN), lambda i: (i, 0)),
        pl.BlockSpec((8 * D_IN, 128), lambda i: (0, 0)),
        pl.BlockSpec((NSC, PB, 128), lambda i: (0, i, 0)),
    ],
    out_specs=[
        pl.BlockSpec((PB, 128), lambda i: (i, 0)),
        pl.BlockSpec((PB, 128), lambda i: (i, 0)),
    ],
    out_shape=[
        jax.ShapeDtypeStruct((NPAD // 8, 128), jnp.float32),
        jax.ShapeDtypeStruct((NPAD // 8, 128), jnp.float32),
    ],
)


def _tc2_body(p_ref, g1_ref, dis_ref, b1_ref, w2_ref, g2_ref):
    s = (p_ref[0] + p_ref[1] + g1_ref[...]) * dis_ref[...] + b1_ref[...]
    h = jnp.maximum(s, 0.0)
    g2_ref[...] = dis_ref[...] * jnp.dot(
        h, w2_ref[...], preferred_element_type=jnp.float32,
        precision=lax.Precision.HIGHEST)


_tc2 = pl.pallas_call(
    _tc2_body,
    grid=(NPAD // RBLK,),
    in_specs=[
        pl.BlockSpec((NSC, PB, 128), lambda i: (0, i, 0)),
        pl.BlockSpec((PB, 128), lambda i: (i, 0)),
        pl.BlockSpec((PB, 128), lambda i: (i, 0)),
        pl.BlockSpec((1, 128), lambda i: (0, 0)),
        pl.BlockSpec((128, 128), lambda i: (0, 0)),
    ],
    out_specs=pl.BlockSpec((PB, 128), lambda i: (i, 0)),
    out_shape=jax.ShapeDtypeStruct((NPAD // 8, 128), jnp.float32),
)


def _tc3_body(q_ref, g2_ref, dis_ref, b2_ref, o_ref):
    o_ref[...] = ((q_ref[0] + q_ref[1] + g2_ref[...]) * dis_ref[...]
                  + b2_ref[...])


_tc3 = pl.pallas_call(
    _tc3_body,
    grid=(NPAD // RBLK,),
    in_specs=[
        pl.BlockSpec((NSC, PB, 128), lambda i: (0, i, 0)),
        pl.BlockSpec((PB, 128), lambda i: (i, 0)),
        pl.BlockSpec((PB, 128), lambda i: (i, 0)),
        pl.BlockSpec((1, 128), lambda i: (0, 0)),
    ],
    out_specs=pl.BlockSpec((PB, 128), lambda i: (i, 0)),
    out_shape=jax.ShapeDtypeStruct((NPAD // 8, 128), jnp.float32),
)


def kernel(x, edge_index, W1, b1, W2, b2):
    x_pad = jnp.pad(x, ((0, NPAD - N), (0, 0)))
    ep = jnp.pad(edge_index, ((0, 0), (0, EPAD - E)), constant_values=N)
    src_e, dst_e = ep[0], ep[1]

    dst_deg0 = dst_e[:BASE1D].reshape(NTILE, NB0D, BATCH)
    dst_deg1 = dst_e[BASE1D:].reshape(NTILE, NB1D, BATCH)
    ones_b = jnp.ones((BATCH,), jnp.float32)
    z1 = jnp.zeros((NPAD,), jnp.float32)
    z2 = jnp.zeros((NPAD, D), jnp.float32)

    x_pk = x_pad.reshape(NPAD // 8, 8 * D_IN)
    w1bd = jnp.kron(jnp.eye(8, dtype=jnp.float32), W1)

    degw = _sc_degree(dst_deg0, dst_deg1, ones_b, z1)
    g1p, disp = _tc1(x_pk, w1bd, degw)
    p = _sc_prop1(g1p.reshape(NPAD, D), src_e, dst_e, z2)

    w2sq = jnp.pad(W2, ((0, 0), (0, D - D_OUT)))
    w2bd = jnp.kron(jnp.eye(8, dtype=jnp.float32), w2sq)
    b1t = jnp.tile(b1, 8).reshape(1, 128)
    b2t = jnp.tile(jnp.pad(b2, (0, D - D_OUT)), 8).reshape(1, 128)

    g2p = _tc2(p.reshape(NSC, NPAD // 8, 128), g1p, disp, b1t, w2bd)
    q = _sc_prop1(g2p.reshape(NPAD, D), src_e, dst_e, z2)
    outp = _tc3(q.reshape(NSC, NPAD // 8, 128), g2p, disp, b2t)
    return outp.reshape(NPAD, D)[:N, :D_OUT]


# speedup vs baseline: 1.0617x; 1.0617x over previous
import functools

import jax
import jax.numpy as jnp
from jax import lax
from jax.experimental import pallas as pl
from jax.experimental.pallas import tpu as pltpu
from jax.experimental.pallas import tpu_sc as plsc

N = 10000
E = 320000
D_IN = 128
D = 16
D2 = 8
D_OUT = 7

NTILE = 16
NSC = 2
NW = NTILE * NSC

NPAD = 10240
RPT = NPAD // NTILE
EPAD = 327680
BATCH = 128
CROWS = 2048
NCH0 = 6
NCH1 = 4
ET0 = NCH0 * CROWS
ET1 = NCH1 * CROWS
BASE1 = NTILE * ET0
NB0 = ET0 // BATCH
NB1 = ET1 // BATCH

_MESH = plsc.VectorSubcoreMesh(core_axis_name="c", subcore_axis_name="s")


NB0D = 96
NB1D = 64
BASE1D = NTILE * NB0D * BATCH


@functools.partial(
    pl.kernel,
    out_type=jax.ShapeDtypeStruct((NSC, NPAD // 8, 128), jnp.float32),
    mesh=_MESH,
    scratch_types=[
        pltpu.VMEM((NB0D, BATCH), jnp.int32),
        pltpu.VMEM((BATCH,), jnp.float32),
        pltpu.VMEM((RPT,), jnp.float32),
        pltpu.VMEM((RPT // 8, 128), jnp.float32),
        pltpu.VMEM_SHARED((NPAD,), jnp.float32),
        pltpu.SemaphoreType.DMA,
        pltpu.SemaphoreType.DMA,
    ],
    compiler_params=pltpu.CompilerParams(needs_layout_passes=False),
)
def _sc_degree(dst0_hbm, dst1_hbm, ones_hbm, z1_hbm, deg_hbm, dst_v, ones_v,
               dv_v, dw_v, deg_s, sem, ssem):
    c = lax.axis_index("c")
    s = lax.axis_index("s")

    pltpu.sync_copy(ones_hbm, ones_v)
    pltpu.sync_copy(z1_hbm.at[pl.ds(s * RPT, RPT)],
                    deg_s.at[pl.ds(s * RPT, RPT)])

    def scatter_deg(nb):
        def issue(g, carry):
            pltpu.async_copy(ones_v, deg_s.at[dst_v.at[g]], ssem, add=True)
            return carry

        lax.fori_loop(0, nb, issue, 0)

        def drain(g, carry):
            pltpu.make_async_copy(ones_v, deg_s.at[dst_v.at[0]], ssem).wait()
            return carry

        lax.fori_loop(0, nb, drain, 0)

    @pl.when(c == 0)
    def _():
        pltpu.async_copy(dst0_hbm.at[s], dst_v, sem).wait()
        plsc.subcore_barrier()
        scatter_deg(NB0D)

    @pl.when(c == 1)
    def _():
        pltpu.async_copy(dst1_hbm.at[s], dst_v.at[pl.ds(0, NB1D)], sem).wait()
        plsc.subcore_barrier()
        scatter_deg(NB1D)

    plsc.subcore_barrier()

    pltpu.sync_copy(deg_s.at[pl.ds(s * RPT, RPT)], dv_v)

    def widen(j, carry):
        idx = jnp.broadcast_to(j, (16,)).astype(jnp.int32)
        dw_v[j // 8, pl.ds((j % 8) * 16, 16)] = plsc.load_gather(dv_v, [idx])
        return carry

    lax.fori_loop(0, RPT, widen, 0, unroll=4)
    pltpu.sync_copy(dw_v, deg_hbm.at[c, pl.ds(s * (RPT // 8), RPT // 8)])


def _make_prop(d):
    @functools.partial(
        pl.kernel,
        out_type=jax.ShapeDtypeStruct((NSC, NPAD, d), jnp.float32),
        mesh=_MESH,
        scratch_types=[
            pltpu.VMEM((NCH0, CROWS), jnp.int32),
            pltpu.VMEM((NCH0, CROWS), jnp.int32),
            pltpu.VMEM((CROWS, d), jnp.float32),
            pltpu.VMEM((CROWS, d), jnp.float32),
            pltpu.VMEM_SHARED((NPAD, d), jnp.float32),
            pltpu.VMEM_SHARED((NPAD, d), jnp.float32),
            pltpu.SemaphoreType.DMA,
            pltpu.SemaphoreType.DMA,
            pltpu.SemaphoreType.DMA,
        ],
        compiler_params=pltpu.CompilerParams(use_tc_tiling_on_sc=False),
    )
    def _sc_prop(g_hbm, src_hbm, dst_hbm, z2_hbm, p_hbm, src_v, dst_v,
                 rows0_v, rows1_v, acc_s, g_s, sem0, sem1, semg):
        c = lax.axis_index("c")
        s = lax.axis_index("s")

        cpg = pltpu.async_copy(g_hbm.at[pl.ds(s * RPT, RPT)],
                               g_s.at[pl.ds(s * RPT, RPT)], semg)
        pltpu.sync_copy(z2_hbm.at[pl.ds(s * RPT, RPT)],
                        acc_s.at[pl.ds(s * RPT, RPT)])

        rows = (rows0_v, rows1_v)
        sems = (sem0, sem1)

        def run(nch, base):
            for k in range(nch):
                pltpu.async_copy(
                    src_hbm.at[pl.ds(base + k * CROWS, CROWS)],
                    src_v.at[k], sem0)
                pltpu.async_copy(
                    dst_hbm.at[pl.ds(base + k * CROWS, CROWS)],
                    dst_v.at[k], sem1)
            for k in range(nch):
                pltpu.make_async_copy(
                    src_hbm.at[pl.ds(base, CROWS)], src_v.at[0], sem0).wait()
                pltpu.make_async_copy(
                    dst_hbm.at[pl.ds(base, CROWS)], dst_v.at[0], sem1).wait()
            cpg.wait()
            plsc.subcore_barrier()

            pltpu.async_copy(g_s.at[src_v.at[0]], rows0_v, sem0)
            for ci in range(nch):
                if ci + 1 < nch:
                    pltpu.async_copy(g_s.at[src_v.at[ci + 1]],
                                     rows[(ci + 1) % 2], sems[(ci + 1) % 2])
                pltpu.make_async_copy(g_s.at[src_v.at[ci]],
                                      rows[ci % 2], sems[ci % 2]).wait()
                pltpu.sync_copy(rows[ci % 2], acc_s.at[dst_v.at[ci]], add=True)

        @pl.when(c == 0)
        def _():
            run(NCH0, s * ET0)

        @pl.when(c == 1)
        def _():
            run(NCH1, BASE1 + s * ET1)

        plsc.subcore_barrier()

        pltpu.sync_copy(acc_s.at[pl.ds(s * RPT, RPT)],
                        p_hbm.at[c, pl.ds(s * RPT, RPT)])

    return _sc_prop


_sc_prop1 = _make_prop(D)


RBLK = 1024


PB = 128


def _tc1_body(x_ref, w_ref, deg_ref, g_ref, dis_ref):
    d = deg_ref[0] + deg_ref[1] + 1.0
    dis = lax.rsqrt(d)
    h = jnp.dot(x_ref[...], w_ref[...], preferred_element_type=jnp.float32,
                precision=lax.Precision.HIGHEST)
    g_ref[...] = dis * h
    dis_ref[...] = dis


_tc1 = pl.pallas_call(
    _tc1_body,
    grid=(NPAD // RBLK,),
    in_specs=[
        pl.BlockSpec((PB, 8 * D_IN), lambda i: (i, 0)),
        pl.BlockSpec((8 * D_IN, 128), lambda i: (0, 0)),
        pl.BlockSpec((NSC, PB, 128), lambda i: (0, i, 0)),
    ],
    out_specs=[
        pl.BlockSpec((PB, 128), lambda i: (i, 0)),
        pl.BlockSpec((PB, 128), lambda i: (i, 0)),
    ],
    out_shape=[
        jax.ShapeDtypeStruct((NPAD // 8, 128), jnp.float32),
        jax.ShapeDtypeStruct((NPAD // 8, 128), jnp.float32),
    ],
)


def _tc2_body(p_ref, g1_ref, dis_ref, b1_ref, w2_ref, g2_ref):
    s = (p_ref[0] + p_ref[1] + g1_ref[...]) * dis_ref[...] + b1_ref[...]
    h = jnp.maximum(s, 0.0)
    g2_ref[...] = dis_ref[...] * jnp.dot(
        h, w2_ref[...], preferred_element_type=jnp.float32,
        precision=lax.Precision.HIGHEST)


_tc2 = pl.pallas_call(
    _tc2_body,
    grid=(NPAD // RBLK,),
    in_specs=[
        pl.BlockSpec((NSC, PB, 128), lambda i: (0, i, 0)),
        pl.BlockSpec((PB, 128), lambda i: (i, 0)),
        pl.BlockSpec((PB, 128), lambda i: (i, 0)),
        pl.BlockSpec((1, 128), lambda i: (0, 0)),
        pl.BlockSpec((128, 128), lambda i: (0, 0)),
    ],
    out_specs=pl.BlockSpec((PB, 128), lambda i: (i, 0)),
    out_shape=jax.ShapeDtypeStruct((NPAD // 8, 128), jnp.float32),
)


def _tc3_body(q_ref, g2_ref, dis_ref, b2_ref, o_ref):
    o_ref[...] = ((q_ref[0] + q_ref[1] + g2_ref[...]) * dis_ref[...]
                  + b2_ref[...])


_tc3 = pl.pallas_call(
    _tc3_body,
    grid=(NPAD // RBLK,),
    in_specs=[
        pl.BlockSpec((NSC, PB, 128), lambda i: (0, i, 0)),
        pl.BlockSpec((PB, 128), lambda i: (i, 0)),
        pl.BlockSpec((PB, 128), lambda i: (i, 0)),
        pl.BlockSpec((1, 128), lambda i: (0, 0)),
    ],
    out_specs=pl.BlockSpec((PB, 128), lambda i: (i, 0)),
    out_shape=jax.ShapeDtypeStruct((NPAD // 8, 128), jnp.float32),
)


def kernel(x, edge_index, W1, b1, W2, b2):
    x_pad = jnp.pad(x, ((0, NPAD - N), (0, 0)))
    ep = jnp.pad(edge_index, ((0, 0), (0, EPAD - E)), constant_values=N)
    src_e, dst_e = ep[0], ep[1]

    dst_deg0 = dst_e[:BASE1D].reshape(NTILE, NB0D, BATCH)
    dst_deg1 = dst_e[BASE1D:].reshape(NTILE, NB1D, BATCH)
    ones_b = jnp.ones((BATCH,), jnp.float32)
    z1 = jnp.zeros((NPAD,), jnp.float32)
    z2 = jnp.zeros((NPAD, D), jnp.float32)

    x_pk = x_pad.reshape(NPAD // 8, 8 * D_IN)
    w1bd = jnp.kron(jnp.eye(8, dtype=jnp.float32), W1)

    degw = _sc_degree(dst_deg0, dst_deg1, ones_b, z1)
    g1p, disp = _tc1(x_pk, w1bd, degw)
    p = _sc_prop1(g1p.reshape(NPAD, D), src_e, dst_e, z2)

    w2sq = jnp.pad(W2, ((0, 0), (0, D - D_OUT)))
    w2bd = jnp.kron(jnp.eye(8, dtype=jnp.float32), w2sq)
    b1t = jnp.tile(b1, 8).reshape(1, 128)
    b2t = jnp.tile(jnp.pad(b2, (0, D - D_OUT)), 8).reshape(1, 128)

    g2p = _tc2(p.reshape(NSC, NPAD // 8, 128), g1p, disp, b1t, w2bd)
    q = _sc_prop1(g2p.reshape(NPAD, D), src_e, dst_e, z2)
    outp = _tc3(q.reshape(NSC, NPAD // 8, 128), g2p, disp, b2t)
    return outp.reshape(NPAD, D)[:N, :D_OUT]


# speedup vs baseline: 1.0619x; 1.0002x over previous
import functools

import jax
import jax.numpy as jnp
from jax import lax
from jax.experimental import pallas as pl
from jax.experimental.pallas import tpu as pltpu
from jax.experimental.pallas import tpu_sc as plsc

N = 10000
E = 320000
D_IN = 128
D = 16
D_OUT = 7

NTILE = 16
NSC = 2
NW = NTILE * NSC

NPAD = 10240
RPT = NPAD // NTILE
EPAD = 327680
BATCH = 128
CROWS = 2048
NCH0 = 6
NCH1 = 4
ET0 = NCH0 * CROWS
ET1 = NCH1 * CROWS
BASE1 = NTILE * ET0

_MESH = plsc.VectorSubcoreMesh(core_axis_name="c", subcore_axis_name="s")


NB0D = 96
NB1D = 64
BASE1D = NTILE * NB0D * BATCH


@functools.partial(
    pl.kernel,
    out_type=jax.ShapeDtypeStruct((NSC, NPAD // 8, 128), jnp.float32),
    mesh=_MESH,
    scratch_types=[
        pltpu.VMEM((NB0D, BATCH), jnp.int32),
        pltpu.VMEM((BATCH,), jnp.float32),
        pltpu.VMEM((RPT,), jnp.float32),
        pltpu.VMEM((RPT // 8, 128), jnp.float32),
        pltpu.VMEM_SHARED((NPAD,), jnp.float32),
        pltpu.SemaphoreType.DMA,
        pltpu.SemaphoreType.DMA,
    ],
    compiler_params=pltpu.CompilerParams(needs_layout_passes=False),
)
def _sc_degree(dst0_hbm, dst1_hbm, ones_hbm, z1_hbm, deg_hbm, dst_v, ones_v,
               dv_v, dw_v, deg_s, sem, ssem):
    c = lax.axis_index("c")
    s = lax.axis_index("s")

    pltpu.sync_copy(ones_hbm, ones_v)
    pltpu.sync_copy(z1_hbm.at[pl.ds(s * RPT, RPT)],
                    deg_s.at[pl.ds(s * RPT, RPT)])

    def scatter_deg(nb):
        def issue(g, carry):
            pltpu.async_copy(ones_v, deg_s.at[dst_v.at[g]], ssem, add=True)
            return carry

        lax.fori_loop(0, nb, issue, 0)

        def drain(g, carry):
            pltpu.make_async_copy(ones_v, deg_s.at[dst_v.at[0]], ssem).wait()
            return carry

        lax.fori_loop(0, nb, drain, 0)

    @pl.when(c == 0)
    def _():
        pltpu.async_copy(dst0_hbm.at[s], dst_v, sem).wait()
        plsc.subcore_barrier()
        scatter_deg(NB0D)

    @pl.when(c == 1)
    def _():
        pltpu.async_copy(dst1_hbm.at[s], dst_v.at[pl.ds(0, NB1D)], sem).wait()
        plsc.subcore_barrier()
        scatter_deg(NB1D)

    plsc.subcore_barrier()

    pltpu.sync_copy(deg_s.at[pl.ds(s * RPT, RPT)], dv_v)

    def widen(j, carry):
        idx = jnp.broadcast_to(j, (16,)).astype(jnp.int32)
        dw_v[j // 8, pl.ds((j % 8) * 16, 16)] = plsc.load_gather(dv_v, [idx])
        return carry

    lax.fori_loop(0, RPT, widen, 0, unroll=4)
    pltpu.sync_copy(dw_v, deg_hbm.at[c, pl.ds(s * (RPT // 8), RPT // 8)])


def _make_prop(d):
    @functools.partial(
        pl.kernel,
        out_type=jax.ShapeDtypeStruct((NSC, NPAD, d), jnp.float32),
        mesh=_MESH,
        scratch_types=[
            pltpu.VMEM((NCH0, CROWS), jnp.int32),
            pltpu.VMEM((NCH0, CROWS), jnp.int32),
            pltpu.VMEM((CROWS, d), jnp.float32),
            pltpu.VMEM((CROWS, d), jnp.float32),
            pltpu.VMEM_SHARED((NPAD, d), jnp.float32),
            pltpu.VMEM_SHARED((NPAD, d), jnp.float32),
            pltpu.SemaphoreType.DMA,
            pltpu.SemaphoreType.DMA,
            pltpu.SemaphoreType.DMA,
        ],
        compiler_params=pltpu.CompilerParams(use_tc_tiling_on_sc=False),
    )
    def _sc_prop(g_hbm, src_hbm, dst_hbm, z2_hbm, p_hbm, src_v, dst_v,
                 rows0_v, rows1_v, acc_s, g_s, sem0, sem1, semg):
        c = lax.axis_index("c")
        s = lax.axis_index("s")

        cpg = pltpu.async_copy(g_hbm.at[pl.ds(s * RPT, RPT)],
                               g_s.at[pl.ds(s * RPT, RPT)], semg)
        pltpu.sync_copy(z2_hbm.at[pl.ds(s * RPT, RPT)],
                        acc_s.at[pl.ds(s * RPT, RPT)])

        rows = (rows0_v, rows1_v)
        sems = (sem0, sem1)

        def run(nch, base):
            for k in range(nch):
                pltpu.async_copy(
                    src_hbm.at[pl.ds(base + k * CROWS, CROWS)],
                    src_v.at[k], sem0)
                pltpu.async_copy(
                    dst_hbm.at[pl.ds(base + k * CROWS, CROWS)],
                    dst_v.at[k], sem1)
            for k in range(nch):
                pltpu.make_async_copy(
                    src_hbm.at[pl.ds(base, CROWS)], src_v.at[0], sem0).wait()
                pltpu.make_async_copy(
                    dst_hbm.at[pl.ds(base, CROWS)], dst_v.at[0], sem1).wait()
            cpg.wait()
            plsc.subcore_barrier()

            pltpu.async_copy(g_s.at[src_v.at[0]], rows0_v, sem0)
            for ci in range(nch):
                if ci + 1 < nch:
                    pltpu.async_copy(g_s.at[src_v.at[ci + 1]],
                                     rows[(ci + 1) % 2], sems[(ci + 1) % 2])
                pltpu.make_async_copy(g_s.at[src_v.at[ci]],
                                      rows[ci % 2], sems[ci % 2]).wait()
                pltpu.sync_copy(rows[ci % 2], acc_s.at[dst_v.at[ci]], add=True)

        @pl.when(c == 0)
        def _():
            run(NCH0, s * ET0)

        @pl.when(c == 1)
        def _():
            run(NCH1, BASE1 + s * ET1)

        plsc.subcore_barrier()

        pltpu.sync_copy(acc_s.at[pl.ds(s * RPT, RPT)],
                        p_hbm.at[c, pl.ds(s * RPT, RPT)])

    return _sc_prop


_sc_prop1 = _make_prop(D)


RBLK = 1024


PB = 128


def _tc1_body(x_ref, w_ref, deg_ref, g_ref, dis_ref):
    d = deg_ref[0] + deg_ref[1] + 1.0
    dis = lax.rsqrt(d)
    h = jnp.dot(x_ref[...], w_ref[...], preferred_element_type=jnp.float32,
                precision=lax.Precision.HIGHEST)
    g_ref[...] = dis * h
    dis_ref[...] = dis


_tc1 = pl.pallas_call(
    _tc1_body,
    grid=(NPAD // RBLK,),
    in_specs=[
        pl.BlockSpec((PB, 8 * D_IN), lambda i: (i, 0)),
        pl.BlockSpec((8 * D_IN, 128), lambda i: (0, 0)),
        pl.BlockSpec((NSC, PB, 128), lambda i: (0, i, 0)),
    ],
    out_specs=[
        pl.BlockSpec((PB, 128), lambda i: (i, 0)),
        pl.BlockSpec((PB, 128), lambda i: (i, 0)),
    ],
    out_shape=[
        jax.ShapeDtypeStruct((NPAD // 8, 128), jnp.float32),
        jax.ShapeDtypeStruct((NPAD // 8, 128), jnp.float32),
    ],
)


def _tc2_body(p_ref, g1_ref, dis_ref, b1_ref, w2_ref, g2_ref):
    s = (p_ref[0] + p_ref[1] + g1_ref[...]) * dis_ref[...] + b1_ref[...]
    h = jnp.maximum(s, 0.0)
    g2_ref[...] = dis_ref[...] * jnp.dot(
        h, w2_ref[...], preferred_element_type=jnp.float32,
        precision=lax.Precision.HIGHEST)


_tc2 = pl.pallas_call(
    _tc2_body,
    grid=(NPAD // RBLK,),
    in_specs=[
        pl.BlockSpec((NSC, PB, 128), lambda i: (0, i, 0)),
        pl.BlockSpec((PB, 128), lambda i: (i, 0)),
        pl.BlockSpec((PB, 128), lambda i: (i, 0)),
        pl.BlockSpec((1, 128), lambda i: (0, 0)),
        pl.BlockSpec((128, 128), lambda i: (0, 0)),
    ],
    out_specs=pl.BlockSpec((PB, 128), lambda i: (i, 0)),
    out_shape=jax.ShapeDtypeStruct((NPAD // 8, 128), jnp.float32),
)


def _tc3_body(q_ref, g2_ref, dis_ref, b2_ref, o_ref):
    o_ref[...] = ((q_ref[0] + q_ref[1] + g2_ref[...]) * dis_ref[...]
                  + b2_ref[...])


_tc3 = pl.pallas_call(
    _tc3_body,
    grid=(NPAD // RBLK,),
    in_specs=[
        pl.BlockSpec((NSC, PB, 128), lambda i: (0, i, 0)),
        pl.BlockSpec((PB, 128), lambda i: (i, 0)),
        pl.BlockSpec((PB, 128), lambda i: (i, 0)),
        pl.BlockSpec((1, 128), lambda i: (0, 0)),
    ],
    out_specs=pl.BlockSpec((PB, 128), lambda i: (i, 0)),
    out_shape=jax.ShapeDtypeStruct((NPAD // 8, 128), jnp.float32),
)


def kernel(x, edge_index, W1, b1, W2, b2):
    x_pad = jnp.pad(x, ((0, NPAD - N), (0, 0)))
    ep = jnp.pad(edge_index, ((0, 0), (0, EPAD - E)), constant_values=N)
    src_e, dst_e = ep[0], ep[1]

    dst_deg0 = dst_e[:BASE1D].reshape(NTILE, NB0D, BATCH)
    dst_deg1 = dst_e[BASE1D:].reshape(NTILE, NB1D, BATCH)
    ones_b = jnp.ones((BATCH,), jnp.float32)
    z1 = jnp.zeros((NPAD,), jnp.float32)
    z2 = jnp.zeros((NPAD, D), jnp.float32)

    x_pk = x_pad.reshape(NPAD // 8, 8 * D_IN)
    w1bd = jnp.kron(jnp.eye(8, dtype=jnp.float32), W1)

    degw = _sc_degree(dst_deg0, dst_deg1, ones_b, z1)
    g1p, disp = _tc1(x_pk, w1bd, degw)
    p = _sc_prop1(g1p.reshape(NPAD, D), src_e, dst_e, z2)

    w2sq = jnp.pad(W2, ((0, 0), (0, D - D_OUT)))
    w2bd = jnp.kron(jnp.eye(8, dtype=jnp.float32), w2sq)
    b1t = jnp.tile(b1, 8).reshape(1, 128)
    b2t = jnp.tile(jnp.pad(b2, (0, D - D_OUT)), 8).reshape(1, 128)

    g2p = _tc2(p.reshape(NSC, NPAD // 8, 128), g1p, disp, b1t, w2bd)
    q = _sc_prop1(g2p.reshape(NPAD, D), src_e, dst_e, z2)
    outp = _tc3(q.reshape(NSC, NPAD // 8, 128), g2p, disp, b2t)
    return outp.reshape(NPAD, D)[:N, :D_OUT]
